# baseline (speedup 1.0000x reference)
import functools

import jax
import jax.numpy as jnp
import numpy as np
from jax import lax
from jax.experimental import pallas as pl
from jax.experimental.pallas import tpu as pltpu
from jax.experimental.pallas import tpu_sc as plsc

B, C, H, W = 16, 768, 32, 32
RANGES = (0.1, 0.3)

NC, NS = 2, 16
NWORK = NC * NS
NPAIR = B // 2
QPP = NWORK // NPAIR

CT = C // 128
BPI = H * (W // 8) * CT
SPAN = BPI // QPP
CH = 12
NCHUNK = SPAN // CH
NSLOT = 4
NBLK = B * BPI


def _static_plan():
    rng = np.random.RandomState(0)
    mix = []
    for i in range(0, B - 1, 2):
        frac = rng.rand() * (RANGES[1] - RANGES[0]) + RANGES[0]
        num_first = int(C * frac)
        perm = rng.permutation(C)
        mix.append(perm[:num_first].copy())
    num_first = int(C * (rng.rand() * (RANGES[1] - RANGES[0]) + RANGES[0]))
    num_second = int(C * (rng.rand() * (RANGES[1] - RANGES[0]) + RANGES[0]))
    perm = rng.permutation(C)
    drop = perm[num_first:num_first + num_second].copy()

    keep = np.ones(C, np.float32)
    keep[drop] = 0.0
    w0 = np.tile(keep, (NPAIR, 1))
    w1 = np.zeros((NPAIR, C), np.float32)
    for k, chans in enumerate(mix):
        w1[k, chans] = keep[chans]
        w0[k, chans] = 0.0
    return (w0.reshape(NPAIR, CT, 128),
            w1.reshape(NPAIR, CT, 128),
            keep.reshape(CT, 128))


_W0, _W1, _KEEP = _static_plan()


def _body(a, w0_h, w1_h, kp_h, out, w0_v, w1_v, kp_v,
          be0, bo0, be1, bo1, be2, bo2, be3, bo3,
          si0, si1, si2, si3, so0, so1, so2, so3):
    wid = lax.axis_index("s") * NC + lax.axis_index("c")
    k = wid // QPP
    q = wid % QPP
    ebase = (2 * k) * BPI + q * SPAN
    obase = ebase + BPI

    pltpu.sync_copy(w0_h.at[k], w0_v)
    pltpu.sync_copy(w1_h.at[k], w1_v)
    pltpu.sync_copy(kp_h, kp_v)

    bes = (be0, be1, be2, be3)
    bos = (bo0, bo1, bo2, bo3)
    sis = (si0, si1, si2, si3)
    sos = (so0, so1, so2, so3)

    def start_in(j, p):
        off = j * CH
        pltpu.async_copy(a.at[pl.ds(ebase + off, CH)], bes[p], sis[p])
        pltpu.async_copy(a.at[pl.ds(obase + off, CH)], bos[p], sis[p])

    def wait_in(p):
        pltpu.make_async_copy(a.at[pl.ds(ebase, CH)], bes[p], sis[p]).wait()
        pltpu.make_async_copy(a.at[pl.ds(obase, CH)], bos[p], sis[p]).wait()

    def start_out(j, p):
        off = j * CH
        pltpu.async_copy(bes[p], out.at[pl.ds(ebase + off, CH)], sos[p])
        pltpu.async_copy(bos[p], out.at[pl.ds(obase + off, CH)], sos[p])

    def wait_out(p):
        pltpu.make_async_copy(bes[p], out.at[pl.ds(ebase, CH)], sos[p]).wait()
        pltpu.make_async_copy(bos[p], out.at[pl.ds(obase, CH)], sos[p]).wait()

    def compute(p):
        be, bo = bes[p], bos[p]

        @plsc.parallel_loop(0, CH * 8, 1, unroll=6)
        def blk_body(i, be=be, bo=bo):
            m = lax.div(i, 8)
            l = lax.rem(i, 8)
            ct = lax.rem(m, CT)
            sl = pl.ds(l * 16, 16)
            w0v = w0_v[ct, sl]
            w1v = w1_v[ct, sl]
            kv = kp_v[ct, sl]
            for s in range(8):
                ve = be[m, s, sl]
                vo = bo[m, s, sl]
                be[m, s, sl] = ve * w0v + vo * w1v
                bo[m, s, sl] = vo * kv

    start_in(0, 0)
    start_in(1, 1)
    start_in(2, 2)

    def ring(jj, _):
        for s in range(NSLOT):
            t = jj * NSLOT + s
            wait_in(s)
            compute(s)
            start_out(t, s)
            r = (s - 1) % NSLOT
            if s == 0:
                @pl.when(jj == 0)
                def _first_fill():
                    start_in(NSLOT - 1, NSLOT - 1)

                @pl.when(jj > 0)
                def _refill0(t=t):
                    wait_out(r)
                    start_in(t + NSLOT - 1, r)
            else:
                @pl.when(t + NSLOT - 1 < NCHUNK)
                def _refill(t=t, s=s, r=r):
                    wait_out(r)
                    start_in(t + NSLOT - 1, r)
        return _

    lax.fori_loop(0, NCHUNK // NSLOT, ring, None)
    for s in range(NSLOT):
        wait_out(s)


def kernel(y):
    a6 = (y.transpose(0, 2, 3, 1)
           .reshape(B, H, W // 8, 8, C // 128, 128)
           .transpose(0, 1, 2, 4, 3, 5))
    a = a6.reshape(NBLK, 8, 128)
    mesh = plsc.VectorSubcoreMesh(core_axis_name="c", subcore_axis_name="s",
                                  num_cores=NC, num_subcores=NS)
    run = pl.kernel(
        _body,
        out_type=jax.ShapeDtypeStruct((NBLK, 8, 128), jnp.float32),
        mesh=mesh,
        scratch_types=(
            [pltpu.VMEM((CT, 128), jnp.float32)] * 3
            + [pltpu.VMEM((CH, 8, 128), jnp.float32)] * (2 * NSLOT)
            + [pltpu.SemaphoreType.DMA] * (2 * NSLOT)
        ),
    )
    o = run(a, jnp.asarray(_W0), jnp.asarray(_W1), jnp.asarray(_KEEP))
    o6 = o.reshape(B, H, W // 8, C // 128, 8, 128)
    return (o6.transpose(0, 1, 2, 4, 3, 5)
              .reshape(B, H, W, C)
              .transpose(0, 3, 1, 2))


# speedup vs baseline: 1.0304x; 1.0304x over previous
import functools

import jax
import jax.numpy as jnp
import numpy as np
from jax import lax
from jax.experimental import pallas as pl
from jax.experimental.pallas import tpu as pltpu
from jax.experimental.pallas import tpu_sc as plsc

B, C, H, W = 16, 768, 32, 32
RANGES = (0.1, 0.3)

NC, NS = 2, 16
NWORK = NC * NS
NPAIR = B // 2
QPP = NWORK // NPAIR

CT = C // 128
BPI = H * (W // 8) * CT
SPAN = BPI // QPP
CH = 16
NCHUNK = SPAN // CH
NSLOT = 3
NBLK = B * BPI


def _static_plan():
    rng = np.random.RandomState(0)
    mix = []
    for i in range(0, B - 1, 2):
        frac = rng.rand() * (RANGES[1] - RANGES[0]) + RANGES[0]
        num_first = int(C * frac)
        perm = rng.permutation(C)
        mix.append(perm[:num_first].copy())
    num_first = int(C * (rng.rand() * (RANGES[1] - RANGES[0]) + RANGES[0]))
    num_second = int(C * (rng.rand() * (RANGES[1] - RANGES[0]) + RANGES[0]))
    perm = rng.permutation(C)
    drop = perm[num_first:num_first + num_second].copy()

    keep = np.ones(C, np.float32)
    keep[drop] = 0.0
    w0 = np.tile(keep, (NPAIR, 1))
    w1 = np.zeros((NPAIR, C), np.float32)
    for k, chans in enumerate(mix):
        w1[k, chans] = keep[chans]
        w0[k, chans] = 0.0
    return (w0.reshape(NPAIR, CT, 128),
            w1.reshape(NPAIR, CT, 128),
            keep.reshape(CT, 128))


_W0, _W1, _KEEP = _static_plan()


def _body(a, w0_h, w1_h, kp_h, out, w0_v, w1_v, kp_v,
          be0, bo0, be1, bo1, be2, bo2,
          si0, si1, si2, so0, so1, so2):
    wid = lax.axis_index("s") * NC + lax.axis_index("c")
    k = wid // QPP
    q = wid % QPP
    ebase = (2 * k) * BPI + q * SPAN
    obase = ebase + BPI

    pltpu.sync_copy(w0_h.at[k], w0_v)
    pltpu.sync_copy(w1_h.at[k], w1_v)
    pltpu.sync_copy(kp_h, kp_v)

    bes = (be0, be1, be2)
    bos = (bo0, bo1, bo2)
    sis = (si0, si1, si2)
    sos = (so0, so1, so2)

    def start_in(j, p):
        off = j * CH
        pltpu.async_copy(a.at[pl.ds(ebase + off, CH)], bes[p], sis[p])
        pltpu.async_copy(a.at[pl.ds(obase + off, CH)], bos[p], sis[p])

    def wait_in(p):
        pltpu.make_async_copy(a.at[pl.ds(ebase, CH)], bes[p], sis[p]).wait()
        pltpu.make_async_copy(a.at[pl.ds(obase, CH)], bos[p], sis[p]).wait()

    def start_out(j, p):
        off = j * CH
        pltpu.async_copy(bes[p], out.at[pl.ds(ebase + off, CH)], sos[p])
        pltpu.async_copy(bos[p], out.at[pl.ds(obase + off, CH)], sos[p])

    def wait_out(p):
        pltpu.make_async_copy(bes[p], out.at[pl.ds(ebase, CH)], sos[p]).wait()
        pltpu.make_async_copy(bos[p], out.at[pl.ds(obase, CH)], sos[p]).wait()

    def compute(p, t):
        be, bo = bes[p], bos[p]
        coff = lax.rem(t * CH, CT)

        @plsc.parallel_loop(0, CH * 8, 1, unroll=4)
        def blk_body(i, be=be, bo=bo):
            m = lax.div(i, 8)
            l = lax.rem(i, 8)
            ct = lax.rem(m + coff, CT)
            sl = pl.ds(l * 16, 16)
            w0v = w0_v[ct, sl]
            w1v = w1_v[ct, sl]
            kv = kp_v[ct, sl]
            for s in range(8):
                ve = be[m, s, sl]
                vo = bo[m, s, sl]
                be[m, s, sl] = ve * w0v + vo * w1v
                bo[m, s, sl] = vo * kv

    for p0 in range(NSLOT - 1):
        start_in(p0, p0)

    def ring(jj, _):
        for s in range(NSLOT):
            t = jj * NSLOT + s
            wait_in(s)
            compute(s, t)
            start_out(t, s)
            r = (s - 1) % NSLOT
            if s == 0:
                @pl.when(jj == 0)
                def _first_fill():
                    start_in(NSLOT - 1, NSLOT - 1)

                @pl.when(jj > 0)
                def _refill0(t=t):
                    wait_out(r)
                    start_in(t + NSLOT - 1, r)
            else:
                @pl.when(t + NSLOT - 1 < NCHUNK)
                def _refill(t=t, s=s, r=r):
                    wait_out(r)
                    start_in(t + NSLOT - 1, r)
        return _

    lax.fori_loop(0, NCHUNK // NSLOT, ring, None)
    for s in range(NSLOT):
        wait_out(s)


def kernel(y):
    a6 = (y.transpose(0, 2, 3, 1)
           .reshape(B, H, W // 8, 8, C // 128, 128)
           .transpose(0, 1, 2, 4, 3, 5))
    a = a6.reshape(NBLK, 8, 128)
    mesh = plsc.VectorSubcoreMesh(core_axis_name="c", subcore_axis_name="s",
                                  num_cores=NC, num_subcores=NS)
    run = pl.kernel(
        _body,
        out_type=jax.ShapeDtypeStruct((NBLK, 8, 128), jnp.float32),
        mesh=mesh,
        scratch_types=(
            [pltpu.VMEM((CT, 128), jnp.float32)] * 3
            + [pltpu.VMEM((CH, 8, 128), jnp.float32)] * (2 * NSLOT)
            + [pltpu.SemaphoreType.DMA] * (2 * NSLOT)
        ),
    )
    o = run(a, jnp.asarray(_W0), jnp.asarray(_W1), jnp.asarray(_KEEP))
    o6 = o.reshape(B, H, W // 8, C // 128, 8, 128)
    return (o6.transpose(0, 1, 2, 4, 3, 5)
              .reshape(B, H, W, C)
              .transpose(0, 3, 1, 2))


# speedup vs baseline: 1.0424x; 1.0116x over previous
import functools

import jax
import jax.numpy as jnp
import numpy as np
from jax import lax
from jax.experimental import pallas as pl
from jax.experimental.pallas import tpu as pltpu
from jax.experimental.pallas import tpu_sc as plsc

B, C, H, W = 16, 768, 32, 32
RANGES = (0.1, 0.3)

NC, NS = 2, 16
NWORK = NC * NS
NPAIR = B // 2
QPP = NWORK // NPAIR

CT = C // 128
BPI = H * (W // 8) * CT
SPAN = BPI // QPP
CH = 12
NCHUNK = SPAN // CH
NSLOT = 4
NBLK = B * BPI


def _static_plan():
    rng = np.random.RandomState(0)
    mix = []
    for i in range(0, B - 1, 2):
        frac = rng.rand() * (RANGES[1] - RANGES[0]) + RANGES[0]
        num_first = int(C * frac)
        perm = rng.permutation(C)
        mix.append(perm[:num_first].copy())
    num_first = int(C * (rng.rand() * (RANGES[1] - RANGES[0]) + RANGES[0]))
    num_second = int(C * (rng.rand() * (RANGES[1] - RANGES[0]) + RANGES[0]))
    perm = rng.permutation(C)
    drop = perm[num_first:num_first + num_second].copy()

    keep = np.ones(C, np.float32)
    keep[drop] = 0.0
    w0 = np.tile(keep, (NPAIR, 1))
    w1 = np.zeros((NPAIR, C), np.float32)
    for k, chans in enumerate(mix):
        w1[k, chans] = keep[chans]
        w0[k, chans] = 0.0
    return (w0.reshape(NPAIR, CT, 128),
            w1.reshape(NPAIR, CT, 128),
            keep.reshape(CT, 128))


_W0, _W1, _KEEP = _static_plan()


def _body(a, w0_h, w1_h, kp_h, out, w0_v, w1_v, kp_v,
          be0, bo0, be1, bo1, be2, bo2, be3, bo3,
          si0, si1, si2, si3, so0, so1, so2, so3):
    wid = lax.axis_index("s") * NC + lax.axis_index("c")
    k = wid // QPP
    q = wid % QPP
    ebase = (2 * k) * BPI + q * SPAN
    obase = ebase + BPI

    pltpu.sync_copy(w0_h.at[k], w0_v)
    pltpu.sync_copy(w1_h.at[k], w1_v)
    pltpu.sync_copy(kp_h, kp_v)

    bes = (be0, be1, be2, be3)
    bos = (bo0, bo1, bo2, bo3)
    sis = (si0, si1, si2, si3)
    sos = (so0, so1, so2, so3)

    def start_in(j, p):
        off = j * CH
        pltpu.async_copy(a.at[pl.ds(ebase + off, CH)], bes[p], sis[p])
        pltpu.async_copy(a.at[pl.ds(obase + off, CH)], bos[p], sis[p])

    def wait_in(p):
        pltpu.make_async_copy(a.at[pl.ds(ebase, CH)], bes[p], sis[p]).wait()
        pltpu.make_async_copy(a.at[pl.ds(obase, CH)], bos[p], sis[p]).wait()

    def start_out(j, p):
        off = j * CH
        pltpu.async_copy(bes[p], out.at[pl.ds(ebase + off, CH)], sos[p])
        pltpu.async_copy(bos[p], out.at[pl.ds(obase + off, CH)], sos[p])

    def wait_out(p):
        pltpu.make_async_copy(bes[p], out.at[pl.ds(ebase, CH)], sos[p]).wait()
        pltpu.make_async_copy(bos[p], out.at[pl.ds(obase, CH)], sos[p]).wait()

    def compute(p):
        be, bo = bes[p], bos[p]

        @plsc.parallel_loop(0, CH * 8, 1, unroll=4)
        def blk_body(i, be=be, bo=bo):
            m = lax.div(i, 8)
            l = lax.rem(i, 8)
            ct = lax.rem(m, CT)
            sl = pl.ds(l * 16, 16)
            w0v = w0_v[ct, sl]
            w1v = w1_v[ct, sl]
            kv = kp_v[ct, sl]
            for s in range(8):
                ve = be[m, s, sl]
                vo = bo[m, s, sl]
                be[m, s, sl] = ve * w0v + vo * w1v
                bo[m, s, sl] = vo * kv

    start_in(0, 0)
    start_in(1, 1)
    start_in(2, 2)

    def ring(jj, _):
        for s in range(NSLOT):
            t = jj * NSLOT + s
            wait_in(s)
            compute(s)
            start_out(t, s)
            r = (s - 1) % NSLOT
            if s == 0:
                @pl.when(jj == 0)
                def _first_fill():
                    start_in(NSLOT - 1, NSLOT - 1)

                @pl.when(jj > 0)
                def _refill0(t=t):
                    wait_out(r)
                    start_in(t + NSLOT - 1, r)
            else:
                @pl.when(t + NSLOT - 1 < NCHUNK)
                def _refill(t=t, s=s, r=r):
                    wait_out(r)
                    start_in(t + NSLOT - 1, r)
        return _

    lax.fori_loop(0, NCHUNK // NSLOT, ring, None)
    for s in range(NSLOT):
        wait_out(s)


def kernel(y):
    a6 = (y.transpose(0, 2, 3, 1)
           .reshape(B, H, W // 8, 8, C // 128, 128)
           .transpose(0, 1, 2, 4, 3, 5))
    a = a6.reshape(NBLK, 8, 128)
    mesh = plsc.VectorSubcoreMesh(core_axis_name="c", subcore_axis_name="s",
                                  num_cores=NC, num_subcores=NS)
    run = pl.kernel(
        _body,
        out_type=jax.ShapeDtypeStruct((NBLK, 8, 128), jnp.float32),
        mesh=mesh,
        scratch_types=(
            [pltpu.VMEM((CT, 128), jnp.float32)] * 3
            + [pltpu.VMEM((CH, 8, 128), jnp.float32)] * (2 * NSLOT)
            + [pltpu.SemaphoreType.DMA] * (2 * NSLOT)
        ),
    )
    o = run(a, jnp.asarray(_W0), jnp.asarray(_W1), jnp.asarray(_KEEP))
    o6 = o.reshape(B, H, W // 8, C // 128, 8, 128)
    return (o6.transpose(0, 1, 2, 4, 3, 5)
              .reshape(B, H, W, C)
              .transpose(0, 3, 1, 2))
